# baseline (speedup 1.0000x reference)
import functools

import jax
import jax.numpy as jnp
from jax import lax
from jax.experimental import pallas as pl
from jax.experimental.pallas import tpu as pltpu
from jax.experimental.pallas import tpu_sc as plsc

N, T, D = 1024, 65, 768
T_OUT = 17
NUM_WORKERS = 32
N_PER_W = N // NUM_WORKERS
LANES = 16

CH = 4
NGROUPS = N_PER_W // CH
NWIN = 5

WIN_SRC = (0, 16, 24, 32, 40)
WIN_MAP = (
    ((0, 0),),
    tuple((1 + r, 2 + r) for r in range(4)),
    tuple((5 + r, 2 + r) for r in range(4)),
    tuple((9 + r, 2 + r) for r in range(4)),
    tuple((13 + r, 2 + r) for r in range(4)),
)

_mesh = plsc.VectorSubcoreMesh(core_axis_name="c", subcore_axis_name="s")


@functools.partial(
    pl.kernel,
    mesh=_mesh,
    out_type=jax.ShapeDtypeStruct((N, T_OUT, D), jnp.float32),
    scratch_types=[
        pltpu.VMEM((CH, 8, D), jnp.float32),
        pltpu.VMEM((CH, 8, D), jnp.float32),
        pltpu.VMEM((CH, T_OUT, D), jnp.float32),
        [pltpu.SemaphoreType.DMA] * 2,
        pltpu.SemaphoreType.DMA,
    ],
)
def _patch_drop(x_hbm, out_hbm, wslot0, wslot1, obuf, gsems, wsem):
    wid = lax.axis_index("s") * 2 + lax.axis_index("c")
    n0 = wid * N_PER_W
    wslots = (wslot0, wslot1)

    def issue_window(step):
        g, w = divmod(step, NWIN)
        slot = step % 2
        return pltpu.async_copy(
            x_hbm.at[pl.ds(n0 + g * CH, CH), pl.ds(WIN_SRC[w], 8)],
            wslots[slot],
            gsems[slot],
        )

    gh = [issue_window(0), issue_window(1)]
    wh = None
    for g in range(NGROUPS):
        for w in range(NWIN):
            step = g * NWIN + w
            slot = step % 2
            gh[slot].wait()
            if w == 0 and wh is not None:
                wh.wait()
            src = wslots[slot]

            @plsc.parallel_loop(0, D // LANES, unroll=2)
            def _compact(k, w=w, src=src):
                sl = pl.ds(k * LANES, LANES)
                for b in range(CH):
                    for dst_row, src_row in WIN_MAP[w]:
                        obuf[b, dst_row, sl] = src[b, src_row, sl]

            if step + 2 < NGROUPS * NWIN:
                gh[slot] = issue_window(step + 2)
        wh = pltpu.async_copy(
            obuf, out_hbm.at[pl.ds(n0 + g * CH, CH)], wsem
        )
    wh.wait()


def kernel(x):
    return _patch_drop(x)


# speedup vs baseline: 1.1481x; 1.1481x over previous
import functools

import jax
import jax.numpy as jnp
import numpy as np
from jax import lax
from jax.experimental import pallas as pl
from jax.experimental.pallas import tpu as pltpu
from jax.experimental.pallas import tpu_sc as plsc

N, T, D = 1024, 65, 768
T_OUT = 17
NUM_WORKERS = 32
N_PER_W = N // NUM_WORKERS

T_PAD = 18
_GATHER_IDX = np.array(
    [0] + [1 + r * 8 + c for r in range(2, 6) for c in range(1, 5)] + [45],
    dtype=np.int32,
)
_SCATTER_IDX = np.array(list(range(T_OUT)) + [T_OUT - 1], dtype=np.int32)

NSLOTS = 6
LAG = 3

_mesh = plsc.VectorSubcoreMesh(core_axis_name="c", subcore_axis_name="s")


@functools.partial(
    pl.kernel,
    mesh=_mesh,
    out_type=jax.ShapeDtypeStruct((N, T_OUT, D), jnp.float32),
    scratch_types=[
        pltpu.VMEM((T_PAD,), jnp.int32),
        pltpu.VMEM((T_PAD,), jnp.int32),
        pltpu.VMEM((NSLOTS, T_PAD, D), jnp.float32),
        [pltpu.SemaphoreType.DMA] * NSLOTS,
        [pltpu.SemaphoreType.DMA] * NSLOTS,
    ],
)
def _patch_drop(x_hbm, gidx_hbm, sidx_hbm, out_hbm, gidx_v, sidx_v, buf,
                gsems, wsems):
    wid = lax.axis_index("s") * 2 + lax.axis_index("c")
    n0 = wid * N_PER_W
    pltpu.sync_copy(gidx_hbm, gidx_v)
    pltpu.sync_copy(sidx_hbm, sidx_v)
    gh = [None] * NSLOTS
    wh = [None] * NSLOTS
    for i in range(N_PER_W + LAG):
        s = i % NSLOTS
        if i < N_PER_W:
            if wh[s] is not None:
                wh[s].wait()
            gh[s] = pltpu.async_copy(
                x_hbm.at[n0 + i].at[gidx_v], buf.at[s], gsems[s]
            )
        j = i - LAG
        if j >= 0:
            sj = j % NSLOTS
            gh[sj].wait()
            wh[sj] = pltpu.async_copy(
                buf.at[sj], out_hbm.at[n0 + j].at[sidx_v], wsems[sj]
            )
    for s in range(NSLOTS):
        if wh[s] is not None:
            wh[s].wait()


def kernel(x):
    return _patch_drop(x, jnp.asarray(_GATHER_IDX), jnp.asarray(_SCATTER_IDX))
